# baseline (speedup 1.0000x reference)
import functools

import jax
import jax.numpy as jnp
from jax import lax
from jax.experimental import pallas as pl
from jax.experimental.pallas import tpu as pltpu
from jax.experimental.pallas import tpu_sc as plsc

N_NODES = 10000
N_EDGES = 320000
D = 128

NC = 2
NS = 16
NW = NC * NS
EPW = N_EDGES // NW
C = 80
NCHUNK = EPW // C
NI = 4
N_PAD = 10240
RPS = N_PAD // NS


def _sc_body(node_hbm, edge_hbm, src_hbm, dst_hbm, out_hbm,
             sidx0, sidx1, sidx2, sidx3, didx0, didx1, didx2, didx3,
             rows0, rows1, erows0, erows1, acc,
             isem0, isem1, isem2, isem3, gsem0, gsem1,
             esem0, esem1, ssem0, ssem1):
    cid = lax.axis_index("c")
    sid = lax.axis_index("s")
    wid = sid * NC + cid

    sidx = (sidx0, sidx1, sidx2, sidx3)
    didx = (didx0, didx1, didx2, didx3)
    rows = (rows0, rows1)
    erows = (erows0, erows1)
    isem = (isem0, isem1, isem2, isem3)
    gsem = (gsem0, gsem1)
    esem = (esem0, esem1)
    ssem = (ssem0, ssem1)

    def idx_fetch(t, r):
        base = wid * EPW + t * C
        pltpu.async_copy(src_hbm.at[pl.ds(base, C)], sidx[r], isem[r])
        pltpu.async_copy(dst_hbm.at[pl.ds(base, C)], didx[r], isem[r])

    def wait_idx(r):
        pltpu.make_async_copy(src_hbm.at[pl.ds(0, C)], sidx[r],
                              isem[r]).wait()
        pltpu.make_async_copy(dst_hbm.at[pl.ds(0, C)], didx[r],
                              isem[r]).wait()

    def issue_fetch(t, b, r):
        base = wid * EPW + t * C
        pltpu.async_copy(node_hbm.at[sidx[r]], rows[b], gsem[b])
        pltpu.async_copy(edge_hbm.at[pl.ds(base, C), :], erows[b], esem[b])

    def wait_fetch(b):
        pltpu.make_async_copy(node_hbm.at[pl.ds(0, C), :], rows[b],
                              gsem[b]).wait()
        pltpu.make_async_copy(edge_hbm.at[pl.ds(0, C), :], erows[b],
                              esem[b]).wait()

    def issue_scatter(t, b, r):
        pltpu.async_copy(rows[b], acc.at[didx[r]], ssem[b], add=True)
        pltpu.async_copy(erows[b], acc.at[didx[r]], ssem[b], add=True)

    def wait_scatter(b):
        pltpu.make_async_copy(rows[b], acc.at[pl.ds(0, C), :], ssem[b]).wait()
        pltpu.make_async_copy(erows[b], acc.at[pl.ds(0, C), :], ssem[b]).wait()

    idx_fetch(0, 0)
    idx_fetch(1, 1)
    idx_fetch(2, 2)

    z = jnp.zeros((16,), jnp.float32)

    def zero_row(rr, _):
        for j in range(D // 16):
            erows1[rr, pl.ds(j * 16, 16)] = z
        return 0

    lax.fori_loop(0, C, zero_row, 0)
    for k in range(RPS // C):
        pltpu.sync_copy(erows1, acc.at[pl.ds(sid * RPS + k * C, C), :])

    wait_idx(0)
    issue_fetch(0, 0, 0)
    plsc.subcore_barrier()

    def step(t, j, k):
        b = j % 2
        b1 = (j + 1) % 2
        r1 = (j + 1) % NI
        r3 = (j + 3) % NI
        wait_fetch(b)
        issue_scatter(t, b, j % NI)

        if j == 0:
            @pl.when(k > 0)
            def _():
                wait_scatter(b1)
        else:
            wait_scatter(b1)

        if j >= 2:
            @pl.when(k < NCHUNK // NI - 1)
            def _():
                idx_fetch(t + 3, r3)
        else:
            idx_fetch(t + 3, r3)

        wait_idx(r1)
        issue_fetch(t + 1, b1, r1)

    def quad(k, _):
        t0 = NI * k
        for j in range(NI):
            step(t0 + j, j, k)
        return 0

    lax.fori_loop(0, (NCHUNK - 1) // NI, quad, 0)

    wait_fetch(0)
    issue_scatter(NCHUNK - 1, 0, 0)
    wait_scatter(1)
    wait_scatter(0)
    plsc.subcore_barrier()

    pltpu.sync_copy(acc.at[pl.ds(sid * RPS, RPS), :],
                    out_hbm.at[cid, pl.ds(sid * RPS, RPS), :])


@jax.jit
def _sc_aggregate(node_feats, edge_feats, src_flat, dst_flat):
    mesh = plsc.VectorSubcoreMesh(core_axis_name="c", subcore_axis_name="s")
    f = pl.kernel(
        _sc_body,
        out_type=jax.ShapeDtypeStruct((NC, N_PAD, D), jnp.float32),
        mesh=mesh,
        scratch_types=[
            pltpu.VMEM((C,), jnp.int32),
            pltpu.VMEM((C,), jnp.int32),
            pltpu.VMEM((C,), jnp.int32),
            pltpu.VMEM((C,), jnp.int32),
            pltpu.VMEM((C,), jnp.int32),
            pltpu.VMEM((C,), jnp.int32),
            pltpu.VMEM((C,), jnp.int32),
            pltpu.VMEM((C,), jnp.int32),
            pltpu.VMEM((C, D), jnp.float32),
            pltpu.VMEM((C, D), jnp.float32),
            pltpu.VMEM((C, D), jnp.float32),
            pltpu.VMEM((C, D), jnp.float32),
            pltpu.VMEM_SHARED((N_PAD, D), jnp.float32),
        ] + [pltpu.SemaphoreType.DMA] * 10,
    )
    return f(node_feats, edge_feats, src_flat, dst_flat)


def _mlp_body(p_ref, w1_ref, b1_ref, w2_ref, b2_ref, o_ref):
    h = p_ref[0] + p_ref[1]
    a = jnp.dot(h, w1_ref[...], preferred_element_type=jnp.float32)
    a = jnp.maximum(a + b1_ref[...], 0.0)
    o = jnp.dot(a, w2_ref[...], preferred_element_type=jnp.float32)
    o_ref[...] = o + b2_ref[...]


@jax.jit
def _mlp(partials, W1, b1, W2, b2):
    B = 1000
    grid = (N_NODES // B,)
    return pl.pallas_call(
        _mlp_body,
        grid=grid,
        in_specs=[
            pl.BlockSpec((NC, B, D), lambda i: (0, i, 0)),
            pl.BlockSpec((D, 2 * D), lambda i: (0, 0)),
            pl.BlockSpec((1, 2 * D), lambda i: (0, 0)),
            pl.BlockSpec((2 * D, D), lambda i: (0, 0)),
            pl.BlockSpec((1, D), lambda i: (0, 0)),
        ],
        out_specs=pl.BlockSpec((B, D), lambda i: (i, 0)),
        out_shape=jax.ShapeDtypeStruct((N_NODES, D), jnp.float32),
    )(partials, W1, b1, W2, b2)


@jax.jit
def kernel(node_feats, edge_feats, edge_index, W1, b1, W2, b2):
    ei = edge_index.astype(jnp.int32)
    partials = _sc_aggregate(node_feats, edge_feats, ei[0], ei[1])
    return _mlp(partials, W1, b1.reshape(1, -1), W2, b2.reshape(1, -1))


# speedup vs baseline: 1.0108x; 1.0108x over previous
import jax
import jax.numpy as jnp
from jax import lax
from jax.experimental import pallas as pl
from jax.experimental.pallas import tpu as pltpu
from jax.experimental.pallas import tpu_sc as plsc

N_NODES = 10000
N_EDGES = 320000
D = 128

NC = 2
NS = 16
NW = NC * NS
EPW = N_EDGES // NW
C = 80
NCHUNK = EPW // C
NI = 4
N_PAD = 10240
RPS = N_PAD // NS


def _sc_body(node_hbm, edge_hbm, src_hbm, dst_hbm, out_hbm,
             sidx0, sidx1, sidx2, sidx3, didx0, didx1, didx2, didx3,
             rows0, rows1, erows0, erows1, acc,
             isem0, isem1, isem2, isem3, gsem0, gsem1,
             esem0, esem1, ssem0, ssem1):
    cid = lax.axis_index("c")
    sid = lax.axis_index("s")
    wid = sid * NC + cid

    sidx = (sidx0, sidx1, sidx2, sidx3)
    didx = (didx0, didx1, didx2, didx3)
    rows = (rows0, rows1)
    erows = (erows0, erows1)
    isem = (isem0, isem1, isem2, isem3)
    gsem = (gsem0, gsem1)
    esem = (esem0, esem1)
    ssem = (ssem0, ssem1)

    def idx_fetch(t, r):
        base = wid * EPW + t * C
        pltpu.async_copy(src_hbm.at[pl.ds(base, C)], sidx[r], isem[r])
        pltpu.async_copy(dst_hbm.at[pl.ds(base, C)], didx[r], isem[r])

    def wait_idx(r):
        pltpu.make_async_copy(src_hbm.at[pl.ds(0, C)], sidx[r],
                              isem[r]).wait()
        pltpu.make_async_copy(dst_hbm.at[pl.ds(0, C)], didx[r],
                              isem[r]).wait()

    def issue_fetch(t, b, r):
        base = wid * EPW + t * C
        pltpu.async_copy(node_hbm.at[sidx[r]], rows[b], gsem[b])
        pltpu.async_copy(edge_hbm.at[pl.ds(base, C), :], erows[b], esem[b])

    def wait_fetch(b):
        pltpu.make_async_copy(node_hbm.at[pl.ds(0, C), :], rows[b],
                              gsem[b]).wait()
        pltpu.make_async_copy(edge_hbm.at[pl.ds(0, C), :], erows[b],
                              esem[b]).wait()

    def issue_scatter(t, b, r):
        pltpu.async_copy(rows[b], acc.at[didx[r]], ssem[b], add=True)
        pltpu.async_copy(erows[b], acc.at[didx[r]], ssem[b], add=True)

    def wait_scatter(b):
        pltpu.make_async_copy(rows[b], acc.at[pl.ds(0, C), :], ssem[b]).wait()
        pltpu.make_async_copy(erows[b], acc.at[pl.ds(0, C), :], ssem[b]).wait()

    idx_fetch(0, 0)
    idx_fetch(1, 1)
    idx_fetch(2, 2)

    z = jnp.zeros((16,), jnp.float32)

    def zero_row(rr, _):
        for j in range(D // 16):
            erows1[rr, pl.ds(j * 16, 16)] = z
        return 0

    lax.fori_loop(0, C, zero_row, 0)
    for k in range(RPS // C):
        pltpu.sync_copy(erows1, acc.at[pl.ds(sid * RPS + k * C, C), :])

    wait_idx(0)
    issue_fetch(0, 0, 0)
    plsc.subcore_barrier()

    def step(t, j, k):
        b = j % 2
        b1 = (j + 1) % 2
        r1 = (j + 1) % NI
        r3 = (j + 3) % NI
        wait_fetch(b)
        issue_scatter(t, b, j % NI)

        if j == 0:
            @pl.when(k > 0)
            def _():
                wait_scatter(b1)
        else:
            wait_scatter(b1)

        if j >= 2:
            @pl.when(k < NCHUNK // NI - 1)
            def _():
                idx_fetch(t + 3, r3)
        else:
            idx_fetch(t + 3, r3)

        wait_idx(r1)
        issue_fetch(t + 1, b1, r1)

    def quad(k, _):
        t0 = NI * k
        for j in range(NI):
            step(t0 + j, j, k)
        return 0

    lax.fori_loop(0, (NCHUNK - 1) // NI, quad, 0)

    wait_fetch(0)
    issue_scatter(NCHUNK - 1, 0, 0)
    wait_scatter(1)
    wait_scatter(0)
    plsc.subcore_barrier()

    pltpu.sync_copy(acc.at[pl.ds(sid * RPS, RPS), :],
                    out_hbm.at[cid, pl.ds(sid * RPS, RPS), :])


@jax.jit
def _sc_aggregate(node_feats, edge_feats, src_flat, dst_flat):
    mesh = plsc.VectorSubcoreMesh(core_axis_name="c", subcore_axis_name="s")
    f = pl.kernel(
        _sc_body,
        out_type=jax.ShapeDtypeStruct((NC, N_PAD, D), jnp.float32),
        mesh=mesh,
        scratch_types=[
            pltpu.VMEM((C,), jnp.int32),
            pltpu.VMEM((C,), jnp.int32),
            pltpu.VMEM((C,), jnp.int32),
            pltpu.VMEM((C,), jnp.int32),
            pltpu.VMEM((C,), jnp.int32),
            pltpu.VMEM((C,), jnp.int32),
            pltpu.VMEM((C,), jnp.int32),
            pltpu.VMEM((C,), jnp.int32),
            pltpu.VMEM((C, D), jnp.float32),
            pltpu.VMEM((C, D), jnp.float32),
            pltpu.VMEM((C, D), jnp.float32),
            pltpu.VMEM((C, D), jnp.float32),
            pltpu.VMEM_SHARED((N_PAD, D), jnp.float32),
        ] + [pltpu.SemaphoreType.DMA] * 10,
    )
    return f(node_feats, edge_feats, src_flat, dst_flat)


def _mlp_body(p_ref, w1_ref, b1_ref, w2_ref, b2_ref, o_ref):
    h = p_ref[0] + p_ref[1]
    a = jnp.dot(h, w1_ref[...], preferred_element_type=jnp.float32)
    a = jnp.maximum(a + b1_ref[...], 0.0)
    o = jnp.dot(a, w2_ref[...], preferred_element_type=jnp.float32)
    o_ref[...] = o + b2_ref[...]


@jax.jit
def _mlp(partials, W1, b1, W2, b2):
    B = 2000
    grid = (N_NODES // B,)
    return pl.pallas_call(
        _mlp_body,
        grid=grid,
        in_specs=[
            pl.BlockSpec((NC, B, D), lambda i: (0, i, 0)),
            pl.BlockSpec((D, 2 * D), lambda i: (0, 0)),
            pl.BlockSpec((1, 2 * D), lambda i: (0, 0)),
            pl.BlockSpec((2 * D, D), lambda i: (0, 0)),
            pl.BlockSpec((1, D), lambda i: (0, 0)),
        ],
        out_specs=pl.BlockSpec((B, D), lambda i: (i, 0)),
        out_shape=jax.ShapeDtypeStruct((N_NODES, D), jnp.float32),
    )(partials, W1, b1, W2, b2)


@jax.jit
def kernel(node_feats, edge_feats, edge_index, W1, b1, W2, b2):
    ei = edge_index.astype(jnp.int32)
    partials = _sc_aggregate(node_feats, edge_feats, ei[0], ei[1])
    return _mlp(partials, W1, b1.reshape(1, -1), W2, b2.reshape(1, -1))
